# baseline (device time: 121634 ns/iter reference)
import jax
import jax.numpy as jnp
from jax import lax
from jax.experimental import pallas as pl
from jax.experimental.pallas import tpu as pltpu

N_DEV = 16
NSLOT = 6
KSUB = 8
RT = 2 * (N_DEV - 1)

RING = [0, 4, 8, 12, 13, 9, 5, 1, 2, 6, 10, 14, 15, 11, 7, 3]
INV_RING = [0] * N_DEV
for _p, _m in enumerate(RING):
    INV_RING[_m] = _p


def kernel(A, B):
    M, K = A.shape
    _, N = B.shape
    CH = M // N_DEV
    H = N // 2
    CHK = CH // KSUB
    NSUB = NSLOT * KSUB
    SEED = NSUB

    ring = jnp.asarray(RING, dtype=jnp.int32)
    inv = jnp.asarray(INV_RING, dtype=jnp.int32)
    m = lax.axis_index("i").astype(jnp.int32)
    r = inv[m]
    right = ring[(r + 1) % N_DEV]
    left = ring[(r - 1) % N_DEV]
    scalars = [jnp.reshape(v, (1,)) for v in (r, left, right)]

    def body(r_ref, left_ref, right_ref, a_ref, b_ref, out_ref,
             comm_f, comm_b, b_bf, pf, pb,
             send_f, recv_f, send_b, recv_b, credit_f, credit_b):
        r = r_ref[0]
        left = left_ref[0]
        right = right_ref[0]

        def stripe(idx, lo, hi):
            a_blk = a_ref[pl.ds(idx * CH, CH), :].astype(jnp.bfloat16)
            return jnp.dot(
                a_blk, b_bf[:, lo:hi], preferred_element_type=jnp.float32
            )

        def sub(s, j):
            return (s % NSLOT) * KSUB + j

        def mk(fwd, s, j):
            comm = comm_f if fwd else comm_b
            ssem = send_f if fwd else send_b
            rsem = recv_f if fwd else recv_b
            src_idx = SEED + j if s == 0 else sub(s - 1, j)
            return pltpu.make_async_remote_copy(
                src_ref=comm.at[src_idx],
                dst_ref=comm.at[sub(s, j)],
                send_sem=ssem.at[sub(s, j)],
                recv_sem=rsem.at[sub(s, j)],
                device_id=(right if fwd else left,),
                device_id_type=pl.DeviceIdType.MESH,
            )

        b_bf[:, :] = b_ref[:, :].astype(jnp.bfloat16)
        sf = stripe(r, 0, H)
        sb = stripe(r, H, N)
        for j in range(KSUB):
            comm_f[SEED + j, :, :] = (
                sf[j * CHK:(j + 1) * CHK, :].astype(jnp.bfloat16)
            )
            comm_b[SEED + j, :, :] = (
                sb[j * CHK:(j + 1) * CHK, :].astype(jnp.bfloat16)
            )
        barrier_sem = pltpu.get_barrier_semaphore()
        for nbr in (left, right):
            pl.semaphore_signal(
                barrier_sem, inc=1,
                device_id=(nbr,), device_id_type=pl.DeviceIdType.MESH,
            )
        pl.semaphore_wait(barrier_sem, 2)

        sends_f = {}
        sends_b = {}
        for j in range(KSUB):
            sends_f[(0, j)] = mk(True, 0, j)
            sends_b[(0, j)] = mk(False, 0, j)
            sends_f[(0, j)].start()
            sends_b[(0, j)].start()

        pf[0, :, :] = stripe((r - 1) % N_DEV, 0, H)
        pb[0, :, :] = stripe((r + 1) % N_DEV, H, N)

        for s in range(RT):
            if s + 1 <= N_DEV - 2:
                pf[(s + 1) % 2, :, :] = stripe((r - s - 2) % N_DEV, 0, H)
                pb[(s + 1) % 2, :, :] = stripe((r + s + 2) % N_DEV, H, N)
            for j in range(KSUB):
                mk(True, s, j).wait_recv()
                mk(False, s, j).wait_recv()
                if s <= N_DEV - 2:
                    comm_f[sub(s, j), :, :] = (
                        comm_f[sub(s, j), :, :].astype(jnp.float32)
                        + pf[s % 2, pl.ds(j * CHK, CHK), :]
                    ).astype(jnp.bfloat16)
                    comm_b[sub(s, j), :, :] = (
                        comm_b[sub(s, j), :, :].astype(jnp.float32)
                        + pb[s % 2, pl.ds(j * CHK, CHK), :]
                    ).astype(jnp.bfloat16)
                else:
                    t = s - (N_DEV - 1)
                    idx_f = (r - t) % N_DEV
                    idx_b = (r + t) % N_DEV
                    out_ref[pl.ds(idx_f * CH + j * CHK, CHK), 0:H] = (
                        comm_f[sub(s, j), :, :].astype(jnp.float32)
                    )
                    out_ref[pl.ds(idx_b * CH + j * CHK, CHK), H:N] = (
                        comm_b[sub(s, j), :, :].astype(jnp.float32)
                    )
                if s < RT - 1:
                    if j == 0 and s + 1 >= NSLOT:
                        pl.semaphore_wait(credit_f, 1)
                        pl.semaphore_wait(credit_b, 1)
                    if s + 1 >= NSLOT:
                        sends_f[(s + 1 - NSLOT, j)].wait_send()
                        sends_b[(s + 1 - NSLOT, j)].wait_send()
                    nf = mk(True, s + 1, j)
                    nb = mk(False, s + 1, j)
                    nf.start()
                    nb.start()
                    sends_f[(s + 1, j)] = nf
                    sends_b[(s + 1, j)] = nb
            if s == N_DEV - 2:
                own_f = (r + 1) % N_DEV
                own_b = (r - 1) % N_DEV
                for j in range(KSUB):
                    out_ref[pl.ds(own_f * CH + j * CHK, CHK), 0:H] = (
                        comm_f[sub(s, j), :, :].astype(jnp.float32)
                    )
                    out_ref[pl.ds(own_b * CH + j * CHK, CHK), H:N] = (
                        comm_b[sub(s, j), :, :].astype(jnp.float32)
                    )
            if s <= RT - 1 - NSLOT:
                pl.semaphore_signal(
                    credit_f, inc=1,
                    device_id=(left,), device_id_type=pl.DeviceIdType.MESH,
                )
                pl.semaphore_signal(
                    credit_b, inc=1,
                    device_id=(right,), device_id_type=pl.DeviceIdType.MESH,
                )

        for s in range(RT - NSLOT, RT):
            for j in range(KSUB):
                sends_f[(s, j)].wait_send()
                sends_b[(s, j)].wait_send()

    return pl.pallas_call(
        body,
        out_shape=jax.ShapeDtypeStruct((M, N), jnp.float32),
        in_specs=[
            pl.BlockSpec(memory_space=pltpu.SMEM),
            pl.BlockSpec(memory_space=pltpu.SMEM),
            pl.BlockSpec(memory_space=pltpu.SMEM),
            pl.BlockSpec(memory_space=pltpu.VMEM),
            pl.BlockSpec(memory_space=pltpu.VMEM),
        ],
        out_specs=pl.BlockSpec(memory_space=pltpu.VMEM),
        scratch_shapes=[
            pltpu.VMEM((NSUB + KSUB, CHK, H), jnp.bfloat16),
            pltpu.VMEM((NSUB + KSUB, CHK, H), jnp.bfloat16),
            pltpu.VMEM((K, N), jnp.bfloat16),
            pltpu.VMEM((2, CH, H), jnp.float32),
            pltpu.VMEM((2, CH, H), jnp.float32),
            pltpu.SemaphoreType.DMA((NSUB,)),
            pltpu.SemaphoreType.DMA((NSUB,)),
            pltpu.SemaphoreType.DMA((NSUB,)),
            pltpu.SemaphoreType.DMA((NSUB,)),
            pltpu.SemaphoreType.REGULAR,
            pltpu.SemaphoreType.REGULAR,
        ],
        compiler_params=pltpu.CompilerParams(
            collective_id=0,
            vmem_limit_bytes=64 * 1024 * 1024,
        ),
    )(*scalars, A, B)


# device time: 120473 ns/iter; 1.0096x vs baseline; 1.0096x over previous
import jax
import jax.numpy as jnp
from jax import lax
from jax.experimental import pallas as pl
from jax.experimental.pallas import tpu as pltpu

N_DEV = 16
NSLOT = 4
KSUB = 4
RT = 2 * (N_DEV - 1)

RING = [0, 4, 8, 12, 13, 9, 5, 1, 2, 6, 10, 14, 15, 11, 7, 3]
INV_RING = [0] * N_DEV
for _p, _m in enumerate(RING):
    INV_RING[_m] = _p


def kernel(A, B):
    M, K = A.shape
    _, N = B.shape
    CH = M // N_DEV
    H = N // 2
    CHK = CH // KSUB
    NSUB = NSLOT * KSUB
    SEED = NSUB

    ring = jnp.asarray(RING, dtype=jnp.int32)
    inv = jnp.asarray(INV_RING, dtype=jnp.int32)
    m = lax.axis_index("i").astype(jnp.int32)
    r = inv[m]
    right = ring[(r + 1) % N_DEV]
    left = ring[(r - 1) % N_DEV]
    scalars = [jnp.reshape(v, (1,)) for v in (r, left, right)]

    def body(r_ref, left_ref, right_ref, a_ref, b_ref, out_ref,
             comm_f, comm_b, b_bf, pf, pb,
             send_f, recv_f, send_b, recv_b, credit_f, credit_b):
        r = r_ref[0]
        left = left_ref[0]
        right = right_ref[0]

        def stripe(idx, lo, hi):
            a_blk = a_ref[pl.ds(idx * CH, CH), :].astype(jnp.bfloat16)
            return jnp.dot(
                a_blk, b_bf[:, lo:hi], preferred_element_type=jnp.float32
            )

        def sub(s, j):
            return (s % NSLOT) * KSUB + j

        def mk(fwd, s, j):
            comm = comm_f if fwd else comm_b
            ssem = send_f if fwd else send_b
            rsem = recv_f if fwd else recv_b
            src_idx = SEED + j if s == 0 else sub(s - 1, j)
            return pltpu.make_async_remote_copy(
                src_ref=comm.at[src_idx],
                dst_ref=comm.at[sub(s, j)],
                send_sem=ssem.at[sub(s, j)],
                recv_sem=rsem.at[sub(s, j)],
                device_id=(right if fwd else left,),
                device_id_type=pl.DeviceIdType.MESH,
            )

        b_bf[:, :] = b_ref[:, :].astype(jnp.bfloat16)
        sf = stripe(r, 0, H)
        sb = stripe(r, H, N)
        for j in range(KSUB):
            comm_f[SEED + j, :, :] = (
                sf[j * CHK:(j + 1) * CHK, :].astype(jnp.bfloat16)
            )
            comm_b[SEED + j, :, :] = (
                sb[j * CHK:(j + 1) * CHK, :].astype(jnp.bfloat16)
            )
        barrier_sem = pltpu.get_barrier_semaphore()
        for nbr in (left, right):
            pl.semaphore_signal(
                barrier_sem, inc=1,
                device_id=(nbr,), device_id_type=pl.DeviceIdType.MESH,
            )
        pl.semaphore_wait(barrier_sem, 2)

        sends_f = {}
        sends_b = {}
        for j in range(KSUB):
            sends_f[(0, j)] = mk(True, 0, j)
            sends_b[(0, j)] = mk(False, 0, j)
            sends_f[(0, j)].start()
            sends_b[(0, j)].start()

        pf[0, :, :] = stripe((r - 1) % N_DEV, 0, H)
        pb[0, :, :] = stripe((r + 1) % N_DEV, H, N)

        for s in range(RT):
            if s + 1 <= N_DEV - 2:
                pf[(s + 1) % 2, :, :] = stripe((r - s - 2) % N_DEV, 0, H)
                pb[(s + 1) % 2, :, :] = stripe((r + s + 2) % N_DEV, H, N)
            for j in range(KSUB):
                mk(True, s, j).wait_recv()
                mk(False, s, j).wait_recv()
                if s <= N_DEV - 2:
                    comm_f[sub(s, j), :, :] = (
                        comm_f[sub(s, j), :, :].astype(jnp.float32)
                        + pf[s % 2, pl.ds(j * CHK, CHK), :]
                    ).astype(jnp.bfloat16)
                    comm_b[sub(s, j), :, :] = (
                        comm_b[sub(s, j), :, :].astype(jnp.float32)
                        + pb[s % 2, pl.ds(j * CHK, CHK), :]
                    ).astype(jnp.bfloat16)
                else:
                    t = s - (N_DEV - 1)
                    idx_f = (r - t) % N_DEV
                    idx_b = (r + t) % N_DEV
                    out_ref[pl.ds(idx_f * CH + j * CHK, CHK), 0:H] = (
                        comm_f[sub(s, j), :, :].astype(jnp.float32)
                    )
                    out_ref[pl.ds(idx_b * CH + j * CHK, CHK), H:N] = (
                        comm_b[sub(s, j), :, :].astype(jnp.float32)
                    )
                if s < RT - 1:
                    if j == 0 and s + 1 >= NSLOT:
                        pl.semaphore_wait(credit_f, 1)
                        pl.semaphore_wait(credit_b, 1)
                    if s + 1 >= NSLOT:
                        sends_f[(s + 1 - NSLOT, j)].wait_send()
                        sends_b[(s + 1 - NSLOT, j)].wait_send()
                    nf = mk(True, s + 1, j)
                    nb = mk(False, s + 1, j)
                    nf.start()
                    nb.start()
                    sends_f[(s + 1, j)] = nf
                    sends_b[(s + 1, j)] = nb
            if s == N_DEV - 2:
                own_f = (r + 1) % N_DEV
                own_b = (r - 1) % N_DEV
                for j in range(KSUB):
                    out_ref[pl.ds(own_f * CH + j * CHK, CHK), 0:H] = (
                        comm_f[sub(s, j), :, :].astype(jnp.float32)
                    )
                    out_ref[pl.ds(own_b * CH + j * CHK, CHK), H:N] = (
                        comm_b[sub(s, j), :, :].astype(jnp.float32)
                    )
            if s <= RT - 1 - NSLOT:
                pl.semaphore_signal(
                    credit_f, inc=1,
                    device_id=(left,), device_id_type=pl.DeviceIdType.MESH,
                )
                pl.semaphore_signal(
                    credit_b, inc=1,
                    device_id=(right,), device_id_type=pl.DeviceIdType.MESH,
                )

        for s in range(RT - NSLOT, RT):
            for j in range(KSUB):
                sends_f[(s, j)].wait_send()
                sends_b[(s, j)].wait_send()

    return pl.pallas_call(
        body,
        out_shape=jax.ShapeDtypeStruct((M, N), jnp.float32),
        in_specs=[
            pl.BlockSpec(memory_space=pltpu.SMEM),
            pl.BlockSpec(memory_space=pltpu.SMEM),
            pl.BlockSpec(memory_space=pltpu.SMEM),
            pl.BlockSpec(memory_space=pltpu.VMEM),
            pl.BlockSpec(memory_space=pltpu.VMEM),
        ],
        out_specs=pl.BlockSpec(memory_space=pltpu.VMEM),
        scratch_shapes=[
            pltpu.VMEM((NSUB + KSUB, CHK, H), jnp.bfloat16),
            pltpu.VMEM((NSUB + KSUB, CHK, H), jnp.bfloat16),
            pltpu.VMEM((K, N), jnp.bfloat16),
            pltpu.VMEM((2, CH, H), jnp.float32),
            pltpu.VMEM((2, CH, H), jnp.float32),
            pltpu.SemaphoreType.DMA((NSUB,)),
            pltpu.SemaphoreType.DMA((NSUB,)),
            pltpu.SemaphoreType.DMA((NSUB,)),
            pltpu.SemaphoreType.DMA((NSUB,)),
            pltpu.SemaphoreType.REGULAR,
            pltpu.SemaphoreType.REGULAR,
        ],
        compiler_params=pltpu.CompilerParams(
            collective_id=0,
            vmem_limit_bytes=64 * 1024 * 1024,
        ),
    )(*scalars, A, B)
